# baseline (speedup 1.0000x reference)
import functools

import jax
import jax.numpy as jnp
from jax import lax
from jax.experimental import pallas as pl
from jax.experimental.pallas import tpu as pltpu
from jax.experimental.pallas import tpu_sc as plsc

N = 10000
E = 320000
D = 128

NC = 2
NS = 16
L = 16
NW = NC * NS
EP = 327680
EPT = EP // NW
CH = 64
NCHUNK = EPT // CH
NP = 10112
RPT = NP // NS
ZR = RPT // 4
NPD = 10240
DZ = NPD // NS
RB = 1000


def _attn_outputs(h, asr, adr, i, as_ref, ad_ref, ma_ref, md_ref, gb_ref):
    asv = jnp.sum(h * asr, axis=1, keepdims=True)
    adv = jnp.sum(h * adr, axis=1, keepdims=True)
    as_ref[...] = asv
    ad_ref[...] = adv
    cas = jnp.full((1, 1), jnp.max(asv), jnp.float32)
    cad = jnp.full((1, 1), jnp.max(adv), jnp.float32)

    @pl.when(i == 0)
    def _():
        ma_ref[...] = cas
        md_ref[...] = cad

    @pl.when(i > 0)
    def _():
        ma_ref[...] = jnp.maximum(ma_ref[...], cas)
        md_ref[...] = jnp.maximum(md_ref[...], cad)

    @pl.when(i == N // RB - 1)
    def _():
        g = ma_ref[...] + md_ref[...]
        g = jnp.where(g >= 0, g, 0.2 * g)
        gb_ref[...] = jnp.broadcast_to(g, (1, 16))


def _tc_embed_body(x_ref, w_ref, asr_ref, adr_ref, h_ref, as_ref, ad_ref,
                   ma_ref, md_ref, gb_ref):
    h = jnp.dot(x_ref[...], w_ref[...], preferred_element_type=jnp.float32)
    h_ref[...] = h
    _attn_outputs(h, asr_ref[...], adr_ref[...], pl.program_id(0),
                  as_ref, ad_ref, ma_ref, md_ref, gb_ref)


_tc_embed = pl.pallas_call(
    _tc_embed_body,
    grid=(N // RB,),
    in_specs=[
        pl.BlockSpec((RB, D), lambda i: (i, 0)),
        pl.BlockSpec((D, D), lambda i: (0, 0)),
        pl.BlockSpec((1, D), lambda i: (0, 0)),
        pl.BlockSpec((1, D), lambda i: (0, 0)),
    ],
    out_specs=[
        pl.BlockSpec((RB, D), lambda i: (i, 0)),
        pl.BlockSpec((RB, 1), lambda i: (i, 0)),
        pl.BlockSpec((RB, 1), lambda i: (i, 0)),
        pl.BlockSpec((1, 1), lambda i: (0, 0)),
        pl.BlockSpec((1, 1), lambda i: (0, 0)),
        pl.BlockSpec((1, 16), lambda i: (0, 0)),
    ],
    out_shape=[
        jax.ShapeDtypeStruct((N, D), jnp.float32),
        jax.ShapeDtypeStruct((N, 1), jnp.float32),
        jax.ShapeDtypeStruct((N, 1), jnp.float32),
        jax.ShapeDtypeStruct((1, 1), jnp.float32),
        jax.ShapeDtypeStruct((1, 1), jnp.float32),
        jax.ShapeDtypeStruct((1, 16), jnp.float32),
    ],
)


def _tc_mid_body(n0_ref, n1_ref, d0_ref, d1_ref, b_ref, w_ref, asr_ref,
                 adr_ref, h_ref, as_ref, ad_ref, ma_ref, md_ref, gb_ref):
    den = d0_ref[...] + d1_ref[...] + 1e-16
    z = (n0_ref[...] + n1_ref[...]) / den + b_ref[...]
    z = jnp.where(z > 0, z, jnp.exp(jnp.minimum(z, 0.0)) - 1.0)
    h = jnp.dot(z, w_ref[...], preferred_element_type=jnp.float32)
    h_ref[...] = h
    _attn_outputs(h, asr_ref[...], adr_ref[...], pl.program_id(0),
                  as_ref, ad_ref, ma_ref, md_ref, gb_ref)


_tc_mid = pl.pallas_call(
    _tc_mid_body,
    grid=(N // RB,),
    in_specs=[
        pl.BlockSpec((RB, D), lambda i: (i, 0)),
        pl.BlockSpec((RB, D), lambda i: (i, 0)),
        pl.BlockSpec((RB, 1), lambda i: (i, 0)),
        pl.BlockSpec((RB, 1), lambda i: (i, 0)),
        pl.BlockSpec((1, D), lambda i: (0, 0)),
        pl.BlockSpec((D, D), lambda i: (0, 0)),
        pl.BlockSpec((1, D), lambda i: (0, 0)),
        pl.BlockSpec((1, D), lambda i: (0, 0)),
    ],
    out_specs=[
        pl.BlockSpec((RB, D), lambda i: (i, 0)),
        pl.BlockSpec((RB, 1), lambda i: (i, 0)),
        pl.BlockSpec((RB, 1), lambda i: (i, 0)),
        pl.BlockSpec((1, 1), lambda i: (0, 0)),
        pl.BlockSpec((1, 1), lambda i: (0, 0)),
        pl.BlockSpec((1, 16), lambda i: (0, 0)),
    ],
    out_shape=[
        jax.ShapeDtypeStruct((N, D), jnp.float32),
        jax.ShapeDtypeStruct((N, 1), jnp.float32),
        jax.ShapeDtypeStruct((N, 1), jnp.float32),
        jax.ShapeDtypeStruct((1, 1), jnp.float32),
        jax.ShapeDtypeStruct((1, 1), jnp.float32),
        jax.ShapeDtypeStruct((1, 16), jnp.float32),
    ],
)


def _tc_out_body(n0_ref, n1_ref, d0_ref, d1_ref, b_ref, wr_ref, br_ref, y_ref):
    den = d0_ref[...] + d1_ref[...] + 1e-16
    z = (n0_ref[...] + n1_ref[...]) / den + b_ref[...]
    z = jnp.where(z > 0, z, jnp.exp(jnp.minimum(z, 0.0)) - 1.0)
    y_ref[...] = (
        jnp.dot(z, wr_ref[...], preferred_element_type=jnp.float32)
        + br_ref[...]
    )


_tc_out = pl.pallas_call(
    _tc_out_body,
    grid=(N // RB,),
    in_specs=[
        pl.BlockSpec((RB, D), lambda i: (i, 0)),
        pl.BlockSpec((RB, D), lambda i: (i, 0)),
        pl.BlockSpec((RB, 1), lambda i: (i, 0)),
        pl.BlockSpec((RB, 1), lambda i: (i, 0)),
        pl.BlockSpec((1, D), lambda i: (0, 0)),
        pl.BlockSpec((D, 1), lambda i: (0, 0)),
        pl.BlockSpec((1, 1), lambda i: (0, 0)),
    ],
    out_specs=pl.BlockSpec((RB, 1), lambda i: (i, 0)),
    out_shape=jax.ShapeDtypeStruct((N, 1), jnp.float32),
)


NB_I = 4
NB_R = 3


def _sc_gat_body(h_hbm, src_hbm, dst_hbm, as_hbm, ad_hbm, gb_hbm,
                 num_out, den_out,
                 as_v, ad_v, src_v, dst_v, w_v, rows_v, zden_v,
                 gb_v, num_sh, den_sh, gsem, isem, ssem):
    c = lax.axis_index("c")
    s = lax.axis_index("s")
    wid = c * NS + s

    zvec = jnp.zeros((L,), jnp.float32)

    def _zr(i, carry):
        for r in range(D // L):
            rows_v[0, i, pl.ds(r * L, L)] = zvec
        return carry

    lax.fori_loop(0, CH, _zr, 0)

    def _zd(i, carry):
        zden_v[pl.ds(i * L, L)] = zvec
        return carry

    lax.fori_loop(0, DZ // L, _zd, 0)

    for k in range(RPT // CH):
        pltpu.sync_copy(rows_v.at[0], num_sh.at[pl.ds(s * RPT + k * CH, CH)])
    rem = RPT % CH
    pltpu.sync_copy(rows_v.at[0, pl.ds(0, rem)],
                    num_sh.at[pl.ds(s * RPT + (RPT // CH) * CH, rem)])

    pltpu.sync_copy(zden_v, den_sh.at[pl.ds(s * DZ, DZ)])

    pltpu.sync_copy(as_hbm, as_v.at[pl.ds(0, N)])
    pltpu.sync_copy(ad_hbm, ad_v.at[pl.ds(0, N)])
    as_v[pl.ds(N, L)] = zvec
    ad_v[pl.ds(N, L)] = zvec

    pltpu.sync_copy(gb_hbm, gb_v)
    gb = gb_v[pl.ds(0, L)]

    plsc.subcore_barrier()

    ebase = wid * EPT

    def _idx_issue(ci):
        q = ci % NB_I
        base = ebase + ci * CH
        pltpu.async_copy(src_hbm.at[pl.ds(base, CH)], src_v.at[q],
                         isem.at[q])
        pltpu.async_copy(dst_hbm.at[pl.ds(base, CH)], dst_v.at[q],
                         isem.at[q])

    def _idx_wait(ci):
        q = ci % NB_I
        pltpu.make_async_copy(src_hbm.at[pl.ds(0, CH)], src_v.at[q],
                              isem.at[q]).wait()
        pltpu.make_async_copy(dst_hbm.at[pl.ds(0, CH)], dst_v.at[q],
                              isem.at[q]).wait()

    def _gather_issue(ci):
        p = ci % NB_R
        q = ci % NB_I
        pltpu.async_copy(h_hbm.at[src_v.at[q]], rows_v.at[p], gsem.at[p])

    def _gather_wait(ci):
        p = ci % NB_R
        q = ci % NB_I
        pltpu.make_async_copy(h_hbm.at[src_v.at[q]], rows_v.at[p],
                              gsem.at[p]).wait()

    def _scatter_issue(ci):
        p = ci % NB_R
        q = ci % NB_I
        pltpu.async_copy(w_v.at[p], den_sh.at[dst_v.at[q]], ssem.at[p],
                         add=True)
        pltpu.async_copy(rows_v.at[p], num_sh.at[dst_v.at[q]], ssem.at[p],
                         add=True)

    def _scatter_drain(ci):
        p = ci % NB_R
        q = ci % NB_I
        pltpu.make_async_copy(w_v.at[p], den_sh.at[dst_v.at[q]],
                              ssem.at[p]).wait()
        pltpu.make_async_copy(rows_v.at[p], num_sh.at[dst_v.at[q]],
                              ssem.at[p]).wait()

    _idx_issue(0)
    _idx_issue(1)
    _idx_issue(2)
    _idx_wait(0)
    _gather_issue(0)
    _idx_wait(1)
    _gather_issue(1)

    def _step(ci, carry):
        @pl.when(ci > 0)
        def _():
            _scatter_drain(ci - 1)

        @pl.when(ci + 2 < NCHUNK)
        def _():
            _idx_wait(ci + 2)
            _gather_issue(ci + 2)

        @pl.when(ci + 3 < NCHUNK)
        def _():
            _idx_issue(ci + 3)

        _gather_wait(ci)

        p = ci % NB_R
        q = ci % NB_I
        for t in range(CH // L):
            si = src_v[q, pl.ds(t * L, L)]
            di = dst_v[q, pl.ds(t * L, L)]
            e = plsc.load_gather(as_v, [si]) + plsc.load_gather(ad_v, [di])
            e = jnp.where(e >= 0.0, e, 0.2 * e)
            w_v[p, pl.ds(t * L, L)] = jnp.exp(e - gb)

        pv = jnp.full((L,), p, jnp.int32)

        def _row(j, rcarry):
            wsp = plsc.load_gather(w_v, [pv, jnp.full((L,), j, jnp.int32)])
            for r in range(D // L):
                rows_v[p, j, pl.ds(r * L, L)] = (
                    rows_v[p, j, pl.ds(r * L, L)] * wsp)
            return rcarry

        lax.fori_loop(0, CH, _row, 0, unroll=2)
        _scatter_issue(ci)
        return carry

    lax.fori_loop(0, NCHUNK, _step, 0)
    _scatter_drain(NCHUNK - 1)

    plsc.subcore_barrier()

    pltpu.sync_copy(num_sh.at[pl.ds(s * RPT, RPT)],
                    num_out.at[c, pl.ds(s * RPT, RPT)])
    pltpu.sync_copy(den_sh.at[pl.ds(s * DZ, DZ)],
                    den_out.at[pl.ds(c * NPD + s * DZ, DZ)])


def _make_sc_gat():
    mesh = plsc.VectorSubcoreMesh(
        core_axis_name="c", subcore_axis_name="s", num_cores=NC,
        num_subcores=NS)
    return pl.kernel(
        _sc_gat_body,
        compiler_params=pltpu.CompilerParams(needs_layout_passes=False),
        out_type=(
            jax.ShapeDtypeStruct((NC, NP, D), jnp.float32),
            jax.ShapeDtypeStruct((NC * NPD,), jnp.float32),
        ),
        mesh=mesh,
        scratch_types=[
            pltpu.VMEM((N + L,), jnp.float32),
            pltpu.VMEM((N + L,), jnp.float32),
            pltpu.VMEM((NB_I, CH), jnp.int32),
            pltpu.VMEM((NB_I, CH), jnp.int32),
            pltpu.VMEM((NB_R, CH), jnp.float32),
            pltpu.VMEM((NB_R, CH, D), jnp.float32),
            pltpu.VMEM((DZ,), jnp.float32),
            pltpu.VMEM((L,), jnp.float32),
            pltpu.VMEM_SHARED((NP, D), jnp.float32),
            pltpu.VMEM_SHARED((NPD,), jnp.float32),
            pltpu.SemaphoreType.DMA((NB_R,)),
            pltpu.SemaphoreType.DMA((NB_I,)),
            pltpu.SemaphoreType.DMA((NB_R,)),
        ],
    )


def kernel(x, edge_index, W1, a_src1, a_dst1, b1, W2, a_src2, a_dst2, b2,
           Wr, br):
    pad = EP - E
    src = jnp.concatenate([edge_index[0], jnp.zeros((pad,), jnp.int32)])
    dst = jnp.concatenate([edge_index[1], jnp.full((pad,), N, jnp.int32)])

    sc_gat = _make_sc_gat()

    h1, as1, ad1, _, _, gb1 = _tc_embed(x, W1, a_src1, a_dst1)
    num1, den1 = sc_gat(h1, src, dst, as1.reshape(N), ad1.reshape(N),
                        gb1.reshape(16))
    h2, as2, ad2, _, _, gb2 = _tc_mid(
        num1[0, :N, :], num1[1, :N, :],
        den1[0:N].reshape(N, 1), den1[NPD:NPD + N].reshape(N, 1),
        b1.reshape(1, D), W2, a_src2, a_dst2)
    num2, den2 = sc_gat(h2, src, dst, as2.reshape(N), ad2.reshape(N),
                        gb2.reshape(16))
    y = _tc_out(
        num2[0, :N, :], num2[1, :N, :],
        den2[0:N].reshape(N, 1), den2[NPD:NPD + N].reshape(N, 1),
        b2.reshape(1, D), Wr, br.reshape(1, 1))
    return y


# speedup vs baseline: 1.6098x; 1.6098x over previous
import functools

import jax
import jax.numpy as jnp
from jax import lax
from jax.experimental import pallas as pl
from jax.experimental.pallas import tpu as pltpu
from jax.experimental.pallas import tpu_sc as plsc

N = 10000
E = 320000
D = 128

NC = 2
NS = 16
L = 16
NW = NC * NS
EP = 327680
EPT = EP // NW
CH = 64
NCHUNK = EPT // CH
NP = 10112
RPT = NP // NS
ZR = RPT // 4
NPD = 10240
DZ = NPD // NS
RB = 1000


def _attn_outputs(h, asr, adr, i, as_ref, ad_ref, ma_ref, md_ref, gb_ref):
    asv = jnp.sum(h * asr, axis=1, keepdims=True)
    adv = jnp.sum(h * adr, axis=1, keepdims=True)
    as_ref[...] = asv
    ad_ref[...] = adv
    cas = jnp.full((1, 1), jnp.max(asv), jnp.float32)
    cad = jnp.full((1, 1), jnp.max(adv), jnp.float32)

    @pl.when(i == 0)
    def _():
        ma_ref[...] = cas
        md_ref[...] = cad

    @pl.when(i > 0)
    def _():
        ma_ref[...] = jnp.maximum(ma_ref[...], cas)
        md_ref[...] = jnp.maximum(md_ref[...], cad)

    @pl.when(i == N // RB - 1)
    def _():
        g = ma_ref[...] + md_ref[...]
        g = jnp.where(g >= 0, g, 0.2 * g)
        gb_ref[...] = jnp.broadcast_to(g, (1, 16))


def _tc_embed_body(x_ref, w_ref, asr_ref, adr_ref, h_ref, as_ref, ad_ref,
                   ma_ref, md_ref, gb_ref):
    h = jnp.dot(x_ref[...], w_ref[...], preferred_element_type=jnp.float32)
    h_ref[...] = h
    _attn_outputs(h, asr_ref[...], adr_ref[...], pl.program_id(0),
                  as_ref, ad_ref, ma_ref, md_ref, gb_ref)


_tc_embed = pl.pallas_call(
    _tc_embed_body,
    grid=(N // RB,),
    in_specs=[
        pl.BlockSpec((RB, D), lambda i: (i, 0)),
        pl.BlockSpec((D, D), lambda i: (0, 0)),
        pl.BlockSpec((1, D), lambda i: (0, 0)),
        pl.BlockSpec((1, D), lambda i: (0, 0)),
    ],
    out_specs=[
        pl.BlockSpec((RB, D), lambda i: (i, 0)),
        pl.BlockSpec((RB, 1), lambda i: (i, 0)),
        pl.BlockSpec((RB, 1), lambda i: (i, 0)),
        pl.BlockSpec((1, 1), lambda i: (0, 0)),
        pl.BlockSpec((1, 1), lambda i: (0, 0)),
        pl.BlockSpec((1, 16), lambda i: (0, 0)),
    ],
    out_shape=[
        jax.ShapeDtypeStruct((N, D), jnp.float32),
        jax.ShapeDtypeStruct((N, 1), jnp.float32),
        jax.ShapeDtypeStruct((N, 1), jnp.float32),
        jax.ShapeDtypeStruct((1, 1), jnp.float32),
        jax.ShapeDtypeStruct((1, 1), jnp.float32),
        jax.ShapeDtypeStruct((1, 16), jnp.float32),
    ],
)


def _tc_mid_body(n0_ref, n1_ref, d0_ref, d1_ref, b_ref, w_ref, asr_ref,
                 adr_ref, h_ref, as_ref, ad_ref, ma_ref, md_ref, gb_ref):
    den = d0_ref[...] + d1_ref[...] + 1e-16
    z = (n0_ref[...] + n1_ref[...]) / den + b_ref[...]
    z = jnp.where(z > 0, z, jnp.exp(jnp.minimum(z, 0.0)) - 1.0)
    h = jnp.dot(z, w_ref[...], preferred_element_type=jnp.float32)
    h_ref[...] = h
    _attn_outputs(h, asr_ref[...], adr_ref[...], pl.program_id(0),
                  as_ref, ad_ref, ma_ref, md_ref, gb_ref)


_tc_mid = pl.pallas_call(
    _tc_mid_body,
    grid=(N // RB,),
    in_specs=[
        pl.BlockSpec((RB, D), lambda i: (i, 0)),
        pl.BlockSpec((RB, D), lambda i: (i, 0)),
        pl.BlockSpec((RB, 1), lambda i: (i, 0)),
        pl.BlockSpec((RB, 1), lambda i: (i, 0)),
        pl.BlockSpec((1, D), lambda i: (0, 0)),
        pl.BlockSpec((D, D), lambda i: (0, 0)),
        pl.BlockSpec((1, D), lambda i: (0, 0)),
        pl.BlockSpec((1, D), lambda i: (0, 0)),
    ],
    out_specs=[
        pl.BlockSpec((RB, D), lambda i: (i, 0)),
        pl.BlockSpec((RB, 1), lambda i: (i, 0)),
        pl.BlockSpec((RB, 1), lambda i: (i, 0)),
        pl.BlockSpec((1, 1), lambda i: (0, 0)),
        pl.BlockSpec((1, 1), lambda i: (0, 0)),
        pl.BlockSpec((1, 16), lambda i: (0, 0)),
    ],
    out_shape=[
        jax.ShapeDtypeStruct((N, D), jnp.float32),
        jax.ShapeDtypeStruct((N, 1), jnp.float32),
        jax.ShapeDtypeStruct((N, 1), jnp.float32),
        jax.ShapeDtypeStruct((1, 1), jnp.float32),
        jax.ShapeDtypeStruct((1, 1), jnp.float32),
        jax.ShapeDtypeStruct((1, 16), jnp.float32),
    ],
)


def _tc_out_body(n0_ref, n1_ref, d0_ref, d1_ref, b_ref, wr_ref, br_ref, y_ref):
    den = d0_ref[...] + d1_ref[...] + 1e-16
    z = (n0_ref[...] + n1_ref[...]) / den + b_ref[...]
    z = jnp.where(z > 0, z, jnp.exp(jnp.minimum(z, 0.0)) - 1.0)
    y_ref[...] = (
        jnp.dot(z, wr_ref[...], preferred_element_type=jnp.float32)
        + br_ref[...]
    )


_tc_out = pl.pallas_call(
    _tc_out_body,
    grid=(N // RB,),
    in_specs=[
        pl.BlockSpec((RB, D), lambda i: (i, 0)),
        pl.BlockSpec((RB, D), lambda i: (i, 0)),
        pl.BlockSpec((RB, 1), lambda i: (i, 0)),
        pl.BlockSpec((RB, 1), lambda i: (i, 0)),
        pl.BlockSpec((1, D), lambda i: (0, 0)),
        pl.BlockSpec((D, 1), lambda i: (0, 0)),
        pl.BlockSpec((1, 1), lambda i: (0, 0)),
    ],
    out_specs=pl.BlockSpec((RB, 1), lambda i: (i, 0)),
    out_shape=jax.ShapeDtypeStruct((N, 1), jnp.float32),
)


NB_I = 3
NB_R = 2


def _sc_gat_body(h_hbm, src_hbm, dst_hbm, as_hbm, ad_hbm, gb_hbm,
                 num_out, den_out,
                 as_v, ad_v, src_v, dst_v, w_v, rows_v, rowsf_v, zden_v,
                 gb_v, num_sh, den_sh, gsem, isem, ssem):
    c = lax.axis_index("c")
    s = lax.axis_index("s")
    wid = c * NS + s

    zvec = jnp.zeros((L,), jnp.float32)

    def _zr(i, carry):
        for r in range(D // L):
            rowsf_v[0, i, pl.ds(r * L, L)] = zvec
        return carry

    lax.fori_loop(0, CH, _zr, 0)

    def _zd(i, carry):
        zden_v[pl.ds(i * L, L)] = zvec
        return carry

    lax.fori_loop(0, DZ // L, _zd, 0)

    for k in range(RPT // CH):
        pltpu.sync_copy(rowsf_v.at[0], num_sh.at[pl.ds(s * RPT + k * CH, CH)])
    rem = RPT % CH
    pltpu.sync_copy(rowsf_v.at[0, pl.ds(0, rem)],
                    num_sh.at[pl.ds(s * RPT + (RPT // CH) * CH, rem)])

    pltpu.sync_copy(zden_v, den_sh.at[pl.ds(s * DZ, DZ)])

    pltpu.sync_copy(as_hbm, as_v.at[pl.ds(0, N)])
    pltpu.sync_copy(ad_hbm, ad_v.at[pl.ds(0, N)])
    as_v[pl.ds(N, L)] = zvec
    ad_v[pl.ds(N, L)] = zvec

    pltpu.sync_copy(gb_hbm, gb_v)
    gb = gb_v[pl.ds(0, L)]

    plsc.subcore_barrier()

    ebase = wid * EPT

    def _idx_issue(ci):
        q = ci % NB_I
        base = ebase + ci * CH
        pltpu.async_copy(src_hbm.at[pl.ds(base, CH)], src_v.at[q],
                         isem.at[q])
        pltpu.async_copy(dst_hbm.at[pl.ds(base, CH)], dst_v.at[q],
                         isem.at[q])

    def _idx_wait(ci):
        q = ci % NB_I
        pltpu.make_async_copy(src_hbm.at[pl.ds(0, CH)], src_v.at[q],
                              isem.at[q]).wait()
        pltpu.make_async_copy(dst_hbm.at[pl.ds(0, CH)], dst_v.at[q],
                              isem.at[q]).wait()

    def _gather_issue(ci):
        p = ci % NB_R
        q = ci % NB_I
        pltpu.async_copy(h_hbm.at[src_v.at[q]], rows_v.at[p], gsem.at[p])

    def _gather_wait(ci):
        p = ci % NB_R
        q = ci % NB_I
        pltpu.make_async_copy(h_hbm.at[src_v.at[q]], rows_v.at[p],
                              gsem.at[p]).wait()

    def _scatter_issue(ci):
        p = ci % NB_R
        q = ci % NB_I
        pltpu.async_copy(w_v.at[p], den_sh.at[dst_v.at[q]], ssem.at[p],
                         add=True)
        pltpu.async_copy(rowsf_v.at[p], num_sh.at[dst_v.at[q]], ssem.at[p],
                         add=True)

    def _scatter_drain(ci):
        p = ci % NB_R
        q = ci % NB_I
        pltpu.make_async_copy(w_v.at[p], den_sh.at[dst_v.at[q]],
                              ssem.at[p]).wait()
        pltpu.make_async_copy(rowsf_v.at[p], num_sh.at[dst_v.at[q]],
                              ssem.at[p]).wait()

    _idx_issue(0)
    _idx_wait(0)
    _gather_issue(0)
    _idx_issue(1)

    def _step(ci, carry):
        @pl.when(ci > 0)
        def _():
            _scatter_drain(ci - 1)

        @pl.when(ci + 1 < NCHUNK)
        def _():
            _idx_wait(ci + 1)
            _gather_issue(ci + 1)

        @pl.when(ci + 2 < NCHUNK)
        def _():
            _idx_issue(ci + 2)

        _gather_wait(ci)

        p = ci % NB_R
        q = ci % NB_I
        for t in range(CH // L):
            si = src_v[q, pl.ds(t * L, L)]
            di = dst_v[q, pl.ds(t * L, L)]
            e = plsc.load_gather(as_v, [si]) + plsc.load_gather(ad_v, [di])
            e = jnp.where(e >= 0.0, e, 0.2 * e)
            w_v[p, pl.ds(t * L, L)] = jnp.exp(e - gb)

        pv = jnp.full((L,), p, jnp.int32)

        def _row(j, rcarry):
            wsp = plsc.load_gather(w_v, [pv, jnp.full((L,), j, jnp.int32)])
            for r in range(D // (2 * L)):
                w32 = rows_v[p, j, pl.ds(r * L, L)]
                ab = plsc.bitcast(w32, jnp.bfloat16)
                a, b = plsc.unpack(ab, format=plsc.PackFormat.INTERLEAVED)
                rowsf_v[p, j, pl.ds(r * 2 * L, L)] = a * wsp
                rowsf_v[p, j, pl.ds(r * 2 * L + L, L)] = b * wsp
            return rcarry

        lax.fori_loop(0, CH, _row, 0, unroll=2)
        _scatter_issue(ci)
        return carry

    lax.fori_loop(0, NCHUNK, _step, 0)
    _scatter_drain(NCHUNK - 1)

    plsc.subcore_barrier()

    pltpu.sync_copy(num_sh.at[pl.ds(s * RPT, RPT)],
                    num_out.at[c, pl.ds(s * RPT, RPT)])
    pltpu.sync_copy(den_sh.at[pl.ds(s * DZ, DZ)],
                    den_out.at[pl.ds(c * NPD + s * DZ, DZ)])


def _make_sc_gat():
    mesh = plsc.VectorSubcoreMesh(
        core_axis_name="c", subcore_axis_name="s", num_cores=NC,
        num_subcores=NS)
    return pl.kernel(
        _sc_gat_body,
        compiler_params=pltpu.CompilerParams(
            needs_layout_passes=False, use_tc_tiling_on_sc=False),
        out_type=(
            jax.ShapeDtypeStruct((NC, NP, D), jnp.float32),
            jax.ShapeDtypeStruct((NC * NPD,), jnp.float32),
        ),
        mesh=mesh,
        scratch_types=[
            pltpu.VMEM((N + L,), jnp.float32),
            pltpu.VMEM((N + L,), jnp.float32),
            pltpu.VMEM((NB_I, CH), jnp.int32),
            pltpu.VMEM((NB_I, CH), jnp.int32),
            pltpu.VMEM((NB_R, CH), jnp.float32),
            pltpu.VMEM((NB_R, CH, D // 2), jnp.int32),
            pltpu.VMEM((NB_R, CH, D), jnp.float32),
            pltpu.VMEM((DZ,), jnp.float32),
            pltpu.VMEM((L,), jnp.float32),
            pltpu.VMEM_SHARED((NP, D), jnp.float32),
            pltpu.VMEM_SHARED((NPD,), jnp.float32),
            pltpu.SemaphoreType.DMA((NB_R,)),
            pltpu.SemaphoreType.DMA((NB_I,)),
            pltpu.SemaphoreType.DMA((NB_R,)),
        ],
    )


def kernel(x, edge_index, W1, a_src1, a_dst1, b1, W2, a_src2, a_dst2, b2,
           Wr, br):
    pad = EP - E
    src = jnp.concatenate([edge_index[0], jnp.zeros((pad,), jnp.int32)])
    dst = jnp.concatenate([edge_index[1], jnp.full((pad,), N, jnp.int32)])

    sc_gat = _make_sc_gat()

    def _to_bf16_interleaved(h):
        hb = (h.reshape(N, D // 32, 2, L).transpose(0, 1, 3, 2)
              .reshape(N, D // 2, 2).astype(jnp.bfloat16))
        return lax.bitcast_convert_type(hb, jnp.int32)

    h1, as1, ad1, _, _, gb1 = _tc_embed(x, W1, a_src1, a_dst1)
    num1, den1 = sc_gat(_to_bf16_interleaved(h1), src, dst,
                        as1.reshape(N), ad1.reshape(N), gb1.reshape(16))
    h2, as2, ad2, _, _, gb2 = _tc_mid(
        num1[0, :N, :], num1[1, :N, :],
        den1[0:N].reshape(N, 1), den1[NPD:NPD + N].reshape(N, 1),
        b1.reshape(1, D), W2, a_src2, a_dst2)
    num2, den2 = sc_gat(_to_bf16_interleaved(h2), src, dst,
                        as2.reshape(N), ad2.reshape(N), gb2.reshape(16))
    y = _tc_out(
        num2[0, :N, :], num2[1, :N, :],
        den2[0:N].reshape(N, 1), den2[NPD:NPD + N].reshape(N, 1),
        b2.reshape(1, D), Wr, br.reshape(1, 1))
    return y


# speedup vs baseline: 1.6135x; 1.0023x over previous
import functools

import jax
import jax.numpy as jnp
from jax import lax
from jax.experimental import pallas as pl
from jax.experimental.pallas import tpu as pltpu
from jax.experimental.pallas import tpu_sc as plsc

N = 10000
E = 320000
D = 128

NC = 2
NS = 16
L = 16
NW = NC * NS
EP = 327680
EPT = EP // NW
CH = 64
NCHUNK = EPT // CH
NP = 10112
RPT = NP // NS
ZR = RPT // 4
NPD = 10240
DZ = NPD // NS
RB = 1000


def _attn_outputs(h, asr, adr, i, as_ref, ad_ref, ma_ref, md_ref, gb_ref):
    asv = jnp.sum(h * asr, axis=1, keepdims=True)
    adv = jnp.sum(h * adr, axis=1, keepdims=True)
    as_ref[...] = asv
    ad_ref[...] = adv
    cas = jnp.full((1, 1), jnp.max(asv), jnp.float32)
    cad = jnp.full((1, 1), jnp.max(adv), jnp.float32)

    @pl.when(i == 0)
    def _():
        ma_ref[...] = cas
        md_ref[...] = cad

    @pl.when(i > 0)
    def _():
        ma_ref[...] = jnp.maximum(ma_ref[...], cas)
        md_ref[...] = jnp.maximum(md_ref[...], cad)

    @pl.when(i == N // RB - 1)
    def _():
        g = ma_ref[...] + md_ref[...]
        g = jnp.where(g >= 0, g, 0.2 * g)
        gb_ref[...] = jnp.broadcast_to(g, (1, 16))


def _tc_embed_body(x_ref, w_ref, asr_ref, adr_ref, h_ref, as_ref, ad_ref,
                   ma_ref, md_ref, gb_ref):
    h = jnp.dot(x_ref[...], w_ref[...], preferred_element_type=jnp.float32)
    h_ref[...] = h
    _attn_outputs(h, asr_ref[...], adr_ref[...], pl.program_id(0),
                  as_ref, ad_ref, ma_ref, md_ref, gb_ref)


_tc_embed = pl.pallas_call(
    _tc_embed_body,
    grid=(N // RB,),
    in_specs=[
        pl.BlockSpec((RB, D), lambda i: (i, 0)),
        pl.BlockSpec((D, D), lambda i: (0, 0)),
        pl.BlockSpec((1, D), lambda i: (0, 0)),
        pl.BlockSpec((1, D), lambda i: (0, 0)),
    ],
    out_specs=[
        pl.BlockSpec((RB, D), lambda i: (i, 0)),
        pl.BlockSpec((RB, 1), lambda i: (i, 0)),
        pl.BlockSpec((RB, 1), lambda i: (i, 0)),
        pl.BlockSpec((1, 1), lambda i: (0, 0)),
        pl.BlockSpec((1, 1), lambda i: (0, 0)),
        pl.BlockSpec((1, 16), lambda i: (0, 0)),
    ],
    out_shape=[
        jax.ShapeDtypeStruct((N, D), jnp.float32),
        jax.ShapeDtypeStruct((N, 1), jnp.float32),
        jax.ShapeDtypeStruct((N, 1), jnp.float32),
        jax.ShapeDtypeStruct((1, 1), jnp.float32),
        jax.ShapeDtypeStruct((1, 1), jnp.float32),
        jax.ShapeDtypeStruct((1, 16), jnp.float32),
    ],
)


def _tc_mid_body(n0_ref, n1_ref, d0_ref, d1_ref, b_ref, w_ref, asr_ref,
                 adr_ref, h_ref, as_ref, ad_ref, ma_ref, md_ref, gb_ref):
    den = d0_ref[...] + d1_ref[...] + 1e-16
    z = (n0_ref[...] + n1_ref[...]) / den + b_ref[...]
    z = jnp.where(z > 0, z, jnp.exp(jnp.minimum(z, 0.0)) - 1.0)
    h = jnp.dot(z, w_ref[...], preferred_element_type=jnp.float32)
    h_ref[...] = h
    _attn_outputs(h, asr_ref[...], adr_ref[...], pl.program_id(0),
                  as_ref, ad_ref, ma_ref, md_ref, gb_ref)


_tc_mid = pl.pallas_call(
    _tc_mid_body,
    grid=(N // RB,),
    in_specs=[
        pl.BlockSpec((RB, D), lambda i: (i, 0)),
        pl.BlockSpec((RB, D), lambda i: (i, 0)),
        pl.BlockSpec((RB, 1), lambda i: (i, 0)),
        pl.BlockSpec((RB, 1), lambda i: (i, 0)),
        pl.BlockSpec((1, D), lambda i: (0, 0)),
        pl.BlockSpec((D, D), lambda i: (0, 0)),
        pl.BlockSpec((1, D), lambda i: (0, 0)),
        pl.BlockSpec((1, D), lambda i: (0, 0)),
    ],
    out_specs=[
        pl.BlockSpec((RB, D), lambda i: (i, 0)),
        pl.BlockSpec((RB, 1), lambda i: (i, 0)),
        pl.BlockSpec((RB, 1), lambda i: (i, 0)),
        pl.BlockSpec((1, 1), lambda i: (0, 0)),
        pl.BlockSpec((1, 1), lambda i: (0, 0)),
        pl.BlockSpec((1, 16), lambda i: (0, 0)),
    ],
    out_shape=[
        jax.ShapeDtypeStruct((N, D), jnp.float32),
        jax.ShapeDtypeStruct((N, 1), jnp.float32),
        jax.ShapeDtypeStruct((N, 1), jnp.float32),
        jax.ShapeDtypeStruct((1, 1), jnp.float32),
        jax.ShapeDtypeStruct((1, 1), jnp.float32),
        jax.ShapeDtypeStruct((1, 16), jnp.float32),
    ],
)


def _tc_out_body(n0_ref, n1_ref, d0_ref, d1_ref, b_ref, wr_ref, br_ref, y_ref):
    den = d0_ref[...] + d1_ref[...] + 1e-16
    z = (n0_ref[...] + n1_ref[...]) / den + b_ref[...]
    z = jnp.where(z > 0, z, jnp.exp(jnp.minimum(z, 0.0)) - 1.0)
    y_ref[...] = (
        jnp.dot(z, wr_ref[...], preferred_element_type=jnp.float32)
        + br_ref[...]
    )


_tc_out = pl.pallas_call(
    _tc_out_body,
    grid=(N // RB,),
    in_specs=[
        pl.BlockSpec((RB, D), lambda i: (i, 0)),
        pl.BlockSpec((RB, D), lambda i: (i, 0)),
        pl.BlockSpec((RB, 1), lambda i: (i, 0)),
        pl.BlockSpec((RB, 1), lambda i: (i, 0)),
        pl.BlockSpec((1, D), lambda i: (0, 0)),
        pl.BlockSpec((D, 1), lambda i: (0, 0)),
        pl.BlockSpec((1, 1), lambda i: (0, 0)),
    ],
    out_specs=pl.BlockSpec((RB, 1), lambda i: (i, 0)),
    out_shape=jax.ShapeDtypeStruct((N, 1), jnp.float32),
)


NB_I = 3
NB_R = 2


def _sc_gat_body(h_hbm, src_hbm, dst_hbm, as_hbm, ad_hbm, gb_hbm,
                 num_out, den_out,
                 as_v, ad_v, src_v, dst_v, w_v, rows_v, rowsf_v, zden_v,
                 gb_v, num_sh, den_sh, gsem, isem, ssem):
    c = lax.axis_index("c")
    s = lax.axis_index("s")
    wid = c * NS + s

    zvec = jnp.zeros((L,), jnp.float32)

    def _zr(i, carry):
        for r in range(D // L):
            rowsf_v[0, i, pl.ds(r * L, L)] = zvec
        return carry

    lax.fori_loop(0, CH, _zr, 0)

    def _zd(i, carry):
        zden_v[pl.ds(i * L, L)] = zvec
        return carry

    lax.fori_loop(0, DZ // L, _zd, 0)

    for k in range(RPT // CH):
        pltpu.sync_copy(rowsf_v.at[0], num_sh.at[pl.ds(s * RPT + k * CH, CH)])
    rem = RPT % CH
    pltpu.sync_copy(rowsf_v.at[0, pl.ds(0, rem)],
                    num_sh.at[pl.ds(s * RPT + (RPT // CH) * CH, rem)])

    pltpu.sync_copy(zden_v, den_sh.at[pl.ds(s * DZ, DZ)])

    pltpu.sync_copy(as_hbm, as_v.at[pl.ds(0, N)])
    pltpu.sync_copy(ad_hbm, ad_v.at[pl.ds(0, N)])
    as_v[pl.ds(N, L)] = zvec
    ad_v[pl.ds(N, L)] = zvec

    pltpu.sync_copy(gb_hbm, gb_v)
    gb = gb_v[pl.ds(0, L)]

    plsc.subcore_barrier()

    ebase = wid * EPT

    def _idx_issue(ci):
        q = ci % NB_I
        base = ebase + ci * CH
        pltpu.async_copy(src_hbm.at[pl.ds(base, CH)], src_v.at[q],
                         isem.at[q])
        pltpu.async_copy(dst_hbm.at[pl.ds(base, CH)], dst_v.at[q],
                         isem.at[q])

    def _idx_wait(ci):
        q = ci % NB_I
        pltpu.make_async_copy(src_hbm.at[pl.ds(0, CH)], src_v.at[q],
                              isem.at[q]).wait()
        pltpu.make_async_copy(dst_hbm.at[pl.ds(0, CH)], dst_v.at[q],
                              isem.at[q]).wait()

    def _gather_issue(ci):
        p = ci % NB_R
        q = ci % NB_I
        pltpu.async_copy(h_hbm.at[src_v.at[q]], rows_v.at[p], gsem.at[p])

    def _gather_wait(ci):
        p = ci % NB_R
        q = ci % NB_I
        pltpu.make_async_copy(h_hbm.at[src_v.at[q]], rows_v.at[p],
                              gsem.at[p]).wait()

    def _scatter_issue(ci):
        p = ci % NB_R
        q = ci % NB_I
        pltpu.async_copy(w_v.at[p], den_sh.at[dst_v.at[q]], ssem.at[p],
                         add=True)
        pltpu.async_copy(rowsf_v.at[p], num_sh.at[dst_v.at[q]], ssem.at[p],
                         add=True)

    def _scatter_drain(ci):
        p = ci % NB_R
        q = ci % NB_I
        pltpu.make_async_copy(w_v.at[p], den_sh.at[dst_v.at[q]],
                              ssem.at[p]).wait()
        pltpu.make_async_copy(rowsf_v.at[p], num_sh.at[dst_v.at[q]],
                              ssem.at[p]).wait()

    _idx_issue(0)
    _idx_wait(0)
    _gather_issue(0)
    _idx_issue(1)

    def _step(ci, carry):
        @pl.when(ci > 0)
        def _():
            _scatter_drain(ci - 1)

        @pl.when(ci + 1 < NCHUNK)
        def _():
            _idx_wait(ci + 1)
            _gather_issue(ci + 1)

        @pl.when(ci + 2 < NCHUNK)
        def _():
            _idx_issue(ci + 2)

        _gather_wait(ci)

        p = ci % NB_R
        q = ci % NB_I
        for t in range(CH // L):
            si = src_v[q, pl.ds(t * L, L)]
            di = dst_v[q, pl.ds(t * L, L)]
            e = plsc.load_gather(as_v, [si]) + plsc.load_gather(ad_v, [di])
            e = jnp.where(e >= 0.0, e, 0.2 * e)
            w_v[p, pl.ds(t * L, L)] = jnp.exp(e - gb)

        pv = jnp.full((L,), p, jnp.int32)

        def _row(j, rcarry):
            wsp = plsc.load_gather(w_v, [pv, jnp.full((L,), j, jnp.int32)])
            for r in range(D // (2 * L)):
                w32 = rows_v[p, j, pl.ds(r * L, L)]
                ab = plsc.bitcast(w32, jnp.bfloat16)
                a, b = plsc.unpack(ab, format=plsc.PackFormat.INTERLEAVED)
                rowsf_v[p, j, pl.ds(r * 2 * L, L)] = a * wsp
                rowsf_v[p, j, pl.ds(r * 2 * L + L, L)] = b * wsp
            return rcarry

        lax.fori_loop(0, CH, _row, 0, unroll=4)
        _scatter_issue(ci)
        return carry

    lax.fori_loop(0, NCHUNK, _step, 0)
    _scatter_drain(NCHUNK - 1)

    plsc.subcore_barrier()

    pltpu.sync_copy(num_sh.at[pl.ds(s * RPT, RPT)],
                    num_out.at[c, pl.ds(s * RPT, RPT)])
    pltpu.sync_copy(den_sh.at[pl.ds(s * DZ, DZ)],
                    den_out.at[pl.ds(c * NPD + s * DZ, DZ)])


def _make_sc_gat():
    mesh = plsc.VectorSubcoreMesh(
        core_axis_name="c", subcore_axis_name="s", num_cores=NC,
        num_subcores=NS)
    return pl.kernel(
        _sc_gat_body,
        compiler_params=pltpu.CompilerParams(
            needs_layout_passes=False, use_tc_tiling_on_sc=False),
        out_type=(
            jax.ShapeDtypeStruct((NC, NP, D), jnp.float32),
            jax.ShapeDtypeStruct((NC * NPD,), jnp.float32),
        ),
        mesh=mesh,
        scratch_types=[
            pltpu.VMEM((N + L,), jnp.float32),
            pltpu.VMEM((N + L,), jnp.float32),
            pltpu.VMEM((NB_I, CH), jnp.int32),
            pltpu.VMEM((NB_I, CH), jnp.int32),
            pltpu.VMEM((NB_R, CH), jnp.float32),
            pltpu.VMEM((NB_R, CH, D // 2), jnp.int32),
            pltpu.VMEM((NB_R, CH, D), jnp.float32),
            pltpu.VMEM((DZ,), jnp.float32),
            pltpu.VMEM((L,), jnp.float32),
            pltpu.VMEM_SHARED((NP, D), jnp.float32),
            pltpu.VMEM_SHARED((NPD,), jnp.float32),
            pltpu.SemaphoreType.DMA((NB_R,)),
            pltpu.SemaphoreType.DMA((NB_I,)),
            pltpu.SemaphoreType.DMA((NB_R,)),
        ],
    )


def kernel(x, edge_index, W1, a_src1, a_dst1, b1, W2, a_src2, a_dst2, b2,
           Wr, br):
    pad = EP - E
    src = jnp.concatenate([edge_index[0], jnp.zeros((pad,), jnp.int32)])
    dst = jnp.concatenate([edge_index[1], jnp.full((pad,), N, jnp.int32)])

    sc_gat = _make_sc_gat()

    def _to_bf16_interleaved(h):
        hb = (h.reshape(N, D // 32, 2, L).transpose(0, 1, 3, 2)
              .reshape(N, D // 2, 2).astype(jnp.bfloat16))
        return lax.bitcast_convert_type(hb, jnp.int32)

    h1, as1, ad1, _, _, gb1 = _tc_embed(x, W1, a_src1, a_dst1)
    num1, den1 = sc_gat(_to_bf16_interleaved(h1), src, dst,
                        as1.reshape(N), ad1.reshape(N), gb1.reshape(16))
    h2, as2, ad2, _, _, gb2 = _tc_mid(
        num1[0, :N, :], num1[1, :N, :],
        den1[0:N].reshape(N, 1), den1[NPD:NPD + N].reshape(N, 1),
        b1.reshape(1, D), W2, a_src2, a_dst2)
    num2, den2 = sc_gat(_to_bf16_interleaved(h2), src, dst,
                        as2.reshape(N), ad2.reshape(N), gb2.reshape(16))
    y = _tc_out(
        num2[0, :N, :], num2[1, :N, :],
        den2[0:N].reshape(N, 1), den2[NPD:NPD + N].reshape(N, 1),
        b2.reshape(1, D), Wr, br.reshape(1, 1))
    return y
